# baseline (speedup 1.0000x reference)
import functools

import jax
import jax.numpy as jnp
from jax import lax
from jax.experimental import pallas as pl
from jax.experimental.pallas import tpu as pltpu
from jax.experimental.pallas import tpu_sc as plsc

VOCAB = 1000000
EMBED = 64
BATCH = 4096
NUM_NEG = 64
LANES = 128
NBUF = 8

_NC = 2
_NS = 16
_NW = _NC * _NS
_BPW = BATCH // _NW
_NPW = NUM_NEG // 8


def _sc_gather(headT, tailT, biasT, head_idx, tail_idx, neg_idx):
  mesh = plsc.VectorSubcoreMesh(core_axis_name="c", subcore_axis_name="s")

  @functools.partial(
      pl.kernel,
      mesh=mesh,
      compiler_params=pltpu.CompilerParams(needs_layout_passes=False),
      out_type=[
          jax.ShapeDtypeStruct((BATCH, EMBED), jnp.float32),
          jax.ShapeDtypeStruct((BATCH, EMBED), jnp.float32),
          jax.ShapeDtypeStruct((BATCH, 16), jnp.float32),
          jax.ShapeDtypeStruct((NUM_NEG, EMBED), jnp.float32),
      ],
      scratch_types=[
          pltpu.VMEM((_BPW + 16,), jnp.int32),
          pltpu.VMEM((_BPW, EMBED), jnp.float32),
          pltpu.VMEM((_BPW, 16), jnp.float32),
          [pltpu.VMEM((EMBED, LANES), jnp.float32) for _ in range(NBUF)],
          [pltpu.VMEM((1, LANES), jnp.float32) for _ in range(NBUF)],
          pltpu.SemaphoreType.DMA,
          pltpu.SemaphoreType.DMA,
      ],
  )
  def k(headT_hbm, tailT_hbm, biasT_hbm, hidx_hbm, tidx_hbm, nidx_hbm,
        head_out, tail_out, bias_out, neg_out,
        idx_v, rows_v, brows_v, chunks, bchunks, sem, bsem):
    wid = lax.axis_index("s") * _NC + lax.axis_index("c")
    base = wid * _BPW

    def gather_table(tabT_hbm, idx_hbm, idx_base, n, out_ref, out_base,
                     with_bias):
      pltpu.sync_copy(idx_hbm.at[pl.ds(idx_base, n)], idx_v.at[pl.ds(0, n)])
      idx_v[pl.ds(n, 16)] = jnp.zeros((16,), dtype=jnp.int32)

      def fire(vec, b):
        off = pl.multiple_of((vec[b] >> 7) * LANES, LANES)
        pltpu.async_copy(tabT_hbm.at[:, pl.ds(off, LANES)], chunks[b], sem)
        if with_bias:
          pltpu.async_copy(biasT_hbm.at[:, pl.ds(off, LANES)], bchunks[b], bsem)

      def drain(b):
        pltpu.make_async_copy(
            tabT_hbm.at[:, pl.ds(0, LANES)], chunks[b], sem).wait()
        if with_bias:
          pltpu.make_async_copy(
              biasT_hbm.at[:, pl.ds(0, LANES)], bchunks[b], bsem).wait()

      vec0 = idx_v[pl.ds(0, 16)]
      for b in range(NBUF):
        fire(vec0, b)

      def group(g, _):
        gbase = g * NBUF
        vec = idx_v[pl.ds(gbase, 16)]
        vecn = idx_v[pl.ds(gbase + NBUF, 16)]
        for b in range(NBUF):
          drain(b)
          lane_idx = jnp.full((16,), vec[b] & (LANES - 1), dtype=jnp.int32)
          for j in range(EMBED // 16):
            row_idx = lax.iota(jnp.int32, 16) + (16 * j)
            vals = plsc.load_gather(chunks[b], [row_idx, lane_idx])
            rows_v[gbase + b, pl.ds(16 * j, 16)] = vals
          if with_bias:
            zero_idx = jnp.zeros((16,), dtype=jnp.int32)
            bvals = plsc.load_gather(bchunks[b], [zero_idx, lane_idx])
            brows_v[gbase + b, pl.ds(0, 16)] = bvals
          fire(vecn, b)
        return 0

      lax.fori_loop(0, n // NBUF, group, 0, unroll=False)
      for b in range(NBUF):
        drain(b)
      pltpu.sync_copy(rows_v.at[pl.ds(0, n)], out_ref.at[pl.ds(out_base, n)])
      if with_bias:
        pltpu.sync_copy(brows_v.at[pl.ds(0, n)],
                        bias_out.at[pl.ds(out_base, n)])

    gather_table(headT_hbm, hidx_hbm, base, _BPW, head_out, base, False)
    gather_table(tailT_hbm, tidx_hbm, base, _BPW, tail_out, base, True)

    @pl.when(wid < 8)
    def _():
      gather_table(tailT_hbm, nidx_hbm, wid * _NPW, _NPW, neg_out,
                   wid * _NPW, False)

  return k(headT, tailT, biasT, head_idx, tail_idx, neg_idx)


_LN2 = 0.6931471805599453


def _tc_body(head_ref, tail_ref, bias_ref, neg_ref, rel_ref, out_ref):
  bias = bias_ref[:, 0:1]
  ex = head_ref[...] + rel_ref[...]
  pos = jnp.sum(tail_ref[...] * ex, axis=1, keepdims=True) + bias
  pos_loss_c = jnp.log(0.5 * (1.0 + jnp.exp(-pos)))
  neg = lax.dot_general(ex, neg_ref[...],
                        dimension_numbers=(((1,), (1,)), ((), ())),
                        preferred_element_type=jnp.float32)
  neg = neg + bias
  neg_loss_c = jnp.sum(jnp.log(0.5 * (1.0 + jnp.exp(neg))), axis=1, keepdims=True)
  out_ref[0, 0] = (jnp.sum(pos_loss_c + neg_loss_c) * (1.0 / BATCH)
                   + (NUM_NEG + 1) * _LN2)


def _tc_loss(head_vec, tail_vec, bias16, neg_vec, relation_vec):
  return pl.pallas_call(
      _tc_body,
      out_shape=jax.ShapeDtypeStruct((1, 1), jnp.float32),
      in_specs=[pl.BlockSpec(memory_space=pltpu.MemorySpace.VMEM)] * 5,
      out_specs=pl.BlockSpec(memory_space=pltpu.MemorySpace.SMEM),
  )(head_vec, tail_vec, bias16, neg_vec, relation_vec)


def kernel(head_table, tail_table, relation_vec, bias_table, batch_idxs, neg_idx):
  head_idx = batch_idxs[:, 0]
  tail_idx = batch_idxs[:, 1]
  head_vec, tail_vec, bias16, neg_vec = _sc_gather(
      head_table.T, tail_table.T, bias_table.T, head_idx, tail_idx, neg_idx)
  loss = _tc_loss(head_vec, tail_vec, bias16, neg_vec, relation_vec)
  return loss[0, 0]


# speedup vs baseline: 1.2185x; 1.2185x over previous
import functools

import jax
import jax.numpy as jnp
from jax import lax
from jax.experimental import pallas as pl
from jax.experimental.pallas import tpu as pltpu
from jax.experimental.pallas import tpu_sc as plsc

VOCAB = 1000000
EMBED = 64
BATCH = 4096
NUM_NEG = 64
LANES = 128
NBUF = 8

_NC = 2
_NS = 16
_NW = _NC * _NS
_BPW = BATCH // _NW
_NPW = NUM_NEG // 8


def _sc_gather(headT, tailT, biasT, head_idx, tail_idx, neg_idx):
  mesh = plsc.VectorSubcoreMesh(core_axis_name="c", subcore_axis_name="s")

  @functools.partial(
      pl.kernel,
      mesh=mesh,
      compiler_params=pltpu.CompilerParams(needs_layout_passes=False),
      out_type=[
          jax.ShapeDtypeStruct((BATCH, EMBED), jnp.float32),
          jax.ShapeDtypeStruct((BATCH, EMBED), jnp.float32),
          jax.ShapeDtypeStruct((BATCH, 16), jnp.float32),
          jax.ShapeDtypeStruct((NUM_NEG, EMBED), jnp.float32),
      ],
      scratch_types=[
          pltpu.VMEM((_BPW + 16,), jnp.int32),
          pltpu.VMEM((_BPW, EMBED), jnp.float32),
          pltpu.VMEM((_BPW, 16), jnp.float32),
          [pltpu.VMEM((EMBED, LANES), jnp.float32) for _ in range(NBUF)],
          [pltpu.VMEM((1, LANES), jnp.float32) for _ in range(NBUF)],
          pltpu.SemaphoreType.DMA,
          pltpu.SemaphoreType.DMA,
      ],
  )
  def k(headT_hbm, tailT_hbm, biasT_hbm, hidx_hbm, tidx_hbm, nidx_hbm,
        head_out, tail_out, bias_out, neg_out,
        idx_v, rows_v, brows_v, chunks, bchunks, sem, bsem):
    wid = lax.axis_index("s") * _NC + lax.axis_index("c")
    base = wid * _BPW

    def gather_table(tabT_hbm, idx_hbm, idx_base, n, out_ref, out_base,
                     with_bias):
      pltpu.sync_copy(idx_hbm.at[pl.ds(idx_base, n)], idx_v.at[pl.ds(0, n)])

      def group(g, _):
        gbase = g * NBUF
        vec = idx_v[pl.ds(gbase, 16)]
        copies, bcopies = [], []
        for b in range(NBUF):
          off = pl.multiple_of((vec[b] >> 7) * LANES, LANES)
          copies.append(pltpu.async_copy(
              tabT_hbm.at[:, pl.ds(off, LANES)], chunks[b], sem))
          if with_bias:
            bcopies.append(pltpu.async_copy(
                biasT_hbm.at[:, pl.ds(off, LANES)], bchunks[b], bsem))
        for b in range(NBUF):
          copies[b].wait()
          lane_idx = jnp.full((16,), vec[b] & (LANES - 1), dtype=jnp.int32)
          for j in range(EMBED // 16):
            row_idx = lax.iota(jnp.int32, 16) + (16 * j)
            vals = plsc.load_gather(chunks[b], [row_idx, lane_idx])
            rows_v[gbase + b, pl.ds(16 * j, 16)] = vals
          if with_bias:
            bcopies[b].wait()
            zero_idx = jnp.zeros((16,), dtype=jnp.int32)
            bvals = plsc.load_gather(bchunks[b], [zero_idx, lane_idx])
            brows_v[gbase + b, pl.ds(0, 16)] = bvals
        return 0

      lax.fori_loop(0, n // NBUF, group, 0, unroll=False)
      pltpu.sync_copy(rows_v.at[pl.ds(0, n)], out_ref.at[pl.ds(out_base, n)])
      if with_bias:
        pltpu.sync_copy(brows_v.at[pl.ds(0, n)],
                        bias_out.at[pl.ds(out_base, n)])

    gather_table(headT_hbm, hidx_hbm, base, _BPW, head_out, base, False)
    gather_table(tailT_hbm, tidx_hbm, base, _BPW, tail_out, base, True)

    @pl.when(wid < 8)
    def _():
      gather_table(tailT_hbm, nidx_hbm, wid * _NPW, _NPW, neg_out,
                   wid * _NPW, False)

  return k(headT, tailT, biasT, head_idx, tail_idx, neg_idx)


_LN2 = 0.6931471805599453


def _tc_body(head_ref, tail_ref, bias_ref, neg_ref, rel_ref, out_ref):
  bias = bias_ref[:, 0:1]
  ex = head_ref[...] + rel_ref[...]
  pos = jnp.sum(tail_ref[...] * ex, axis=1, keepdims=True) + bias
  pos_loss_c = jnp.log(0.5 * (1.0 + jnp.exp(-pos)))
  neg = lax.dot_general(ex, neg_ref[...],
                        dimension_numbers=(((1,), (1,)), ((), ())),
                        preferred_element_type=jnp.float32)
  neg = neg + bias
  neg_loss_c = jnp.sum(jnp.log(0.5 * (1.0 + jnp.exp(neg))), axis=1, keepdims=True)
  out_ref[0, 0] = (jnp.sum(pos_loss_c + neg_loss_c) * (1.0 / BATCH)
                   + (NUM_NEG + 1) * _LN2)


def _tc_loss(head_vec, tail_vec, bias16, neg_vec, relation_vec):
  return pl.pallas_call(
      _tc_body,
      out_shape=jax.ShapeDtypeStruct((1, 1), jnp.float32),
      in_specs=[pl.BlockSpec(memory_space=pltpu.MemorySpace.VMEM)] * 5,
      out_specs=pl.BlockSpec(memory_space=pltpu.MemorySpace.SMEM),
  )(head_vec, tail_vec, bias16, neg_vec, relation_vec)


def kernel(head_table, tail_table, relation_vec, bias_table, batch_idxs, neg_idx):
  head_idx = batch_idxs[:, 0]
  tail_idx = batch_idxs[:, 1]
  head_vec, tail_vec, bias16, neg_vec = _sc_gather(
      head_table.T, tail_table.T, bias_table.T, head_idx, tail_idx, neg_idx)
  loss = _tc_loss(head_vec, tail_vec, bias16, neg_vec, relation_vec)
  return loss[0, 0]
